# baseline (speedup 1.0000x reference)
import jax
import jax.numpy as jnp
from jax.experimental import pallas as pl

_H = 76
_W = 76
_HW = _H * _W
_C = 85
_CA = 255
_A = 3
_STRIDE = 8.0
_DH = 4
_NT = _H // _DH


def _decode_kernel(x_ref, o_ref):
    x = x_ref[...]

    lane = jax.lax.broadcasted_iota(jnp.int32, (1, 1, 1, _CA), 3)
    l85 = lane % _C
    jcol = jax.lax.broadcasted_iota(jnp.int32, (1, _W, 1, 1), 1).astype(jnp.float32)

    aw = jnp.where(lane < _C, 10.0, jnp.where(lane < 2 * _C, 16.0, 33.0))
    ah = jnp.where(lane < _C, 13.0, jnp.where(lane < 2 * _C, 30.0, 23.0))

    sig = jax.nn.sigmoid(x)
    ex = jnp.exp(x)
    y = jnp.where(l85 < 2, (sig + jcol) * _STRIDE,
        jnp.where(l85 == 2, ex * aw,
        jnp.where(l85 == 3, ex * ah, sig)))

    jj = jax.lax.broadcasted_iota(jnp.int32, (1, 1, _W), 2)
    gidx = 4 * (jj % 19)
    rowsel = jj // 19

    chunks = [[], [], []]
    for d in range(_DH):
        yd = y[d]
        t3 = jnp.stack([yd[:, b, :].T for b in range(16)], axis=1)
        for a in range(_A):
            t4 = t3[a * _C:a * _C + 4]
            zrows = []
            for k in range(4):
                g = jnp.take_along_axis(
                    t4, jnp.broadcast_to(gidx + k, (4, 16, _W)), axis=2)
                zk = jnp.where(rowsel == 0, g[0:1],
                     jnp.where(rowsel == 1, g[1:2],
                     jnp.where(rowsel == 2, g[2:3], g[3:4])))
                zrows.append(zk)
            chunks[a].append(jnp.concatenate(
                zrows + [t3[a * _C + 4:(a + 1) * _C]], axis=0))

    per_a = [jnp.concatenate(c, axis=2) for c in chunks]
    o_ref[:, 0] = jnp.stack(per_a, axis=0)


def kernel(inputs):
    B = inputs.shape[0]
    xt = jnp.transpose(inputs, (2, 3, 0, 1))
    out = pl.pallas_call(
        _decode_kernel,
        grid=(_NT,),
        in_specs=[pl.BlockSpec((_DH, _W, B, _CA), lambda t: (t, 0, 0, 0))],
        out_specs=pl.BlockSpec((_A, 1, _C, B, _DH * _W),
                               lambda t: (0, t, 0, 0, 0)),
        out_shape=jax.ShapeDtypeStruct((_A, _NT, _C, B, _DH * _W), jnp.float32),
    )(xt)
    return jnp.transpose(out, (3, 0, 1, 4, 2)).reshape(B, _A * _HW, _C)


# speedup vs baseline: 4.0177x; 4.0177x over previous
import jax
import jax.numpy as jnp
from jax.experimental import pallas as pl

_H = 76
_W = 76
_HW = _H * _W
_C = 85
_A = 3
_ANCHOR_W = (10.0, 16.0, 33.0)
_ANCHOR_H = (13.0, 30.0, 23.0)
_STRIDE = 8.0


def _decode_kernel(x_ref, o_ref):
    a = pl.program_id(0) % _A
    x = x_ref[0]

    jj = jax.lax.broadcasted_iota(jnp.int32, (1, _H, _W), 2)
    gx = jj.astype(jnp.float32)
    aw = jnp.where(a == 0, _ANCHOR_W[0], jnp.where(a == 1, _ANCHOR_W[1], _ANCHOR_W[2]))
    ah = jnp.where(a == 0, _ANCHOR_H[0], jnp.where(a == 1, _ANCHOR_H[1], _ANCHOR_H[2]))

    cx = (jax.nn.sigmoid(x[0:1]) + gx) * _STRIDE
    cy = (jax.nn.sigmoid(x[1:2]) + gx) * _STRIDE
    w = jnp.exp(x[2:3]) * aw
    h = jnp.exp(x[3:4]) * ah
    y4 = jnp.concatenate([cx, cy, w, h], axis=0)

    rowsel = jj // 19
    zrows = []
    for k in range(4):
        idx = jnp.broadcast_to(4 * (jj % 19) + k, (4, _H, _W))
        g = jnp.take_along_axis(y4, idx, axis=2)
        zk = jnp.where(rowsel == 0, g[0:1],
             jnp.where(rowsel == 1, g[1:2],
             jnp.where(rowsel == 2, g[2:3], g[3:4])))
        zrows.append(zk)

    rest = jax.nn.sigmoid(x[4:_C])
    w_all = jnp.concatenate(zrows + [rest], axis=0)

    pieces = [w_all[:, hh, :].T for hh in range(_H)]
    o_ref[0] = jnp.concatenate(pieces, axis=0)


def kernel(inputs):
    B = inputs.shape[0]
    out = pl.pallas_call(
        _decode_kernel,
        grid=(B * _A,),
        in_specs=[pl.BlockSpec((1, _C, _H, _W), lambda i: (i // _A, i % _A, 0, 0))],
        out_specs=pl.BlockSpec((1, _HW, _C), lambda i: (i // _A, i % _A, 0)),
        out_shape=jax.ShapeDtypeStruct((B, _A * _HW, _C), jnp.float32),
    )(inputs)
    return out
